# baseline (speedup 1.0000x reference)
import jax
import jax.numpy as jnp
from jax import lax
from jax.experimental import pallas as pl
from jax.experimental.pallas import tpu as pltpu

B = 256
ND = 1024
NB = 8
NS = 4
NDSM = 128
F32 = jnp.float32


def _branch_expand(coeff):
    lane_branch = lax.broadcasted_iota(jnp.int32, (NB, ND), 1) // NDSM
    row = lax.broadcasted_iota(jnp.int32, (NB, ND), 0)
    E = (lane_branch == row).astype(F32)
    return jnp.dot(coeff, E, preferred_element_type=F32)


def _enc_body(out0_ref, wsw_ref, bsw_ref, w1_ref, b1_ref, w2_ref, b2_ref,
              g_ref, nz_ref, out_ref, coeff_ref, state):
    s = pl.program_id(0)

    @pl.when(s == 0)
    def _():
        state[...] = out0_ref[...]

    x = state[...]
    o = jnp.maximum(x, 0.0)

    yl = jnp.dot(o, wsw_ref[0, 0], preferred_element_type=F32) + bsw_ref[0, 0]
    zm = jnp.dot(o, wsw_ref[0, 1], preferred_element_type=F32) + bsw_ref[0, 1]
    zlv = jnp.dot(o, wsw_ref[0, 2], preferred_element_type=F32) + bsw_ref[0, 2]

    gl = yl + g_ref[0]
    m = jnp.max(gl, axis=1, keepdims=True)
    iota = lax.broadcasted_iota(jnp.int32, (B, NB), 1)
    idx = jnp.min(jnp.where(gl >= m, iota, NB), axis=1, keepdims=True)
    onehot = (iota == idx).astype(F32)
    z = nz_ref[0] * jnp.exp(zlv * 0.5) + zm
    coeff = onehot * z
    coeff_ref[0] = coeff

    h = jnp.maximum(jnp.dot(o, w1_ref[0], preferred_element_type=F32) + b1_ref[0], 0.0)
    hm = h * _branch_expand(coeff)
    sp = jnp.dot(hm, w2_ref[0], preferred_element_type=F32) \
        + jnp.dot(coeff, b2_ref[0], preferred_element_type=F32)
    x = x + sp
    state[...] = x
    out_ref[...] = x


def _mid_body(x_ref, wm_ref, bm_ref, wv_ref, bv_ref, wl_ref, bl_ref, nz_ref,
              out_ref):
    o = jnp.maximum(x_ref[...], 0.0)
    z2m = jnp.dot(o, wm_ref[...], preferred_element_type=F32) + bm_ref[...]
    z2lv = jnp.dot(o, wv_ref[...], preferred_element_type=F32) + bv_ref[...]
    z2 = nz_ref[...] * jnp.exp(z2lv * 0.5) + z2m
    out_ref[...] = jnp.dot(z2, wl_ref[...], preferred_element_type=F32) + bl_ref[...]


def _dec_body(d0_ref, w1_ref, b1_ref, w2_ref, b2_ref, coeff_ref, out_ref, state):
    s = pl.program_id(0)

    @pl.when(s == 0)
    def _():
        state[...] = d0_ref[...]

    x = state[...]
    o = jnp.maximum(x, 0.0)
    coeff = coeff_ref[0]
    h = jnp.maximum(jnp.dot(o, w1_ref[0], preferred_element_type=F32) + b1_ref[0], 0.0)
    hm = h * _branch_expand(coeff)
    sp = jnp.dot(hm, w2_ref[0], preferred_element_type=F32) \
        + jnp.dot(coeff, b2_ref[0], preferred_element_type=F32)
    x = x + sp
    state[...] = x
    out_ref[...] = jnp.maximum(x, 0.0)


def _conv(x, W, b, stride):
    y = lax.conv_general_dilated(x, W, (stride, stride), ((1, 1), (1, 1)),
                                 dimension_numbers=('NCHW', 'OIHW', 'NCHW'))
    return y + b[None, :, None, None]


def _deconv(x, W, b):
    y = lax.conv_general_dilated(x, jnp.flip(W, (2, 3)), (1, 1), ((2, 2), (2, 2)),
                                 lhs_dilation=(2, 2),
                                 dimension_numbers=('NCHW', 'IOHW', 'NCHW'))
    return y + b[None, :, None, None]


def _switch_weights(switches):
    w1t = jnp.stack([p['W1'].reshape(ND, ND).T for p in switches])
    b1 = jnp.stack([p['b1'].reshape(1, ND) for p in switches])
    w2t = jnp.stack([p['W2'].transpose(0, 2, 1).reshape(ND, ND) for p in switches])
    b2 = jnp.stack([p['b2'] for p in switches])
    return w1t, b1, w2t, b2


def kernel(x, params):
    key = jax.random.key(42)
    gs, nzs = [], []
    for i in range(NS):
        kg = jax.random.fold_in(key, 2 * i)
        kn = jax.random.fold_in(key, 2 * i + 1)
        gs.append(-jnp.log(jax.random.exponential(kg, (B, NB)) + 1e-20))
        nzs.append(jax.random.normal(kn, (B, NB)))
    g_all = jnp.stack(gs)
    nz_all = jnp.stack(nzs)
    nz2 = jax.random.normal(jax.random.fold_in(key, 999), (B, 10))

    out = jax.nn.relu(_conv(x, params['c1W'], params['c1b'], 2))
    out = jax.nn.relu(_conv(out, params['c2W'], params['c2b'], 2))
    out = jax.nn.relu(_conv(out, params['c3W'], params['c3b'], 2))
    out = _conv(out, params['c4W'], params['c4b'], 2)
    out0 = out.reshape(B, ND)

    enc = params['enc_switches']
    wsw = jnp.stack([p['Wsw'].reshape(3, NB, ND).transpose(0, 2, 1) for p in enc])
    bsw = jnp.stack([p['bsw'].reshape(3, 1, NB) for p in enc])
    ew1t, eb1, ew2t, eb2 = _switch_weights(enc)
    dw1t, db1, dw2t, db2 = _switch_weights(params['dec_switches'])

    full = lambda *_: (0, 0)
    per_sw = lambda nd: pl.BlockSpec((1,) + nd, lambda s: (s,) + (0,) * len(nd))

    out_enc, coeff_all = pl.pallas_call(
        _enc_body,
        grid=(NS,),
        in_specs=[
            pl.BlockSpec((B, ND), full),
            per_sw((3, ND, NB)),
            per_sw((3, 1, NB)),
            per_sw((ND, ND)),
            per_sw((1, ND)),
            per_sw((ND, ND)),
            per_sw((NB, ND)),
            per_sw((B, NB)),
            per_sw((B, NB)),
        ],
        out_specs=[
            pl.BlockSpec((B, ND), full),
            per_sw((B, NB)),
        ],
        out_shape=[
            jax.ShapeDtypeStruct((B, ND), F32),
            jax.ShapeDtypeStruct((NS, B, NB), F32),
        ],
        scratch_shapes=[pltpu.VMEM((B, ND), F32)],
    )(out0, wsw, bsw, ew1t, eb1, ew2t, eb2, g_all, nz_all)

    d0 = pl.pallas_call(
        _mid_body,
        out_shape=jax.ShapeDtypeStruct((B, ND), F32),
    )(out_enc,
      params['fc_mean_W'].T, params['fc_mean_b'].reshape(1, 10),
      params['fc_logvar_W'].T, params['fc_logvar_b'].reshape(1, 10),
      params['fc_latent_W'].T, params['fc_latent_b'].reshape(1, ND),
      nz2)

    d = pl.pallas_call(
        _dec_body,
        grid=(NS,),
        in_specs=[
            pl.BlockSpec((B, ND), full),
            per_sw((ND, ND)),
            per_sw((1, ND)),
            per_sw((ND, ND)),
            per_sw((NB, ND)),
            per_sw((B, NB)),
        ],
        out_specs=pl.BlockSpec((B, ND), full),
        out_shape=jax.ShapeDtypeStruct((B, ND), F32),
        scratch_shapes=[pltpu.VMEM((B, ND), F32)],
    )(d0, dw1t, db1, dw2t, db2, coeff_all)

    d = d.reshape(B, 64, 4, 4)
    d = jax.nn.relu(_deconv(d, params['d1W'], params['d1b']))
    d = jax.nn.relu(_deconv(d, params['d2W'], params['d2b']))
    d = jax.nn.relu(_deconv(d, params['d3W'], params['d3b']))
    d = _deconv(d, params['d4W'], params['d4b'])
    return d


# speedup vs baseline: 1.5484x; 1.5484x over previous
import jax
import jax.numpy as jnp
from jax import lax
from jax.experimental import pallas as pl
from jax.experimental.pallas import tpu as pltpu

B = 256
ND = 1024
NB = 8
NS = 4
NDSM = 128
F32 = jnp.float32


def _dott(a, w):
    return lax.dot_general(a, w, (((1,), (1,)), ((), ())),
                           preferred_element_type=F32)


def _mlp_sp(o, coeff, w1, b1, w2, b2):
    w1f = w1.reshape(ND, ND)
    h = jnp.maximum(_dott(o, w1f) + b1, 0.0)
    cb = coeff.astype(jnp.bfloat16).astype(F32)
    sp = None
    for n in range(NB):
        on = _dott(h[:, n * NDSM:(n + 1) * NDSM], w2[n]) + b2[n:n + 1]
        on = on.astype(jnp.bfloat16).astype(F32)
        term = cb[:, n:n + 1] * on
        sp = term if sp is None else sp + term
    return sp


def _mega_body(*refs):
    (out0_ref, wsw_ref, bsw_ref, b1e_ref, b2e_ref, b1d_ref, b2d_ref,
     g_ref, nz_ref, nz2_ref, wm_ref, bm_ref, wv_ref, bv_ref, wl_ref, bl_ref) = refs[:16]
    w_hbm = refs[16:16 + 16]
    out_ref = refs[32]
    w1buf, w2buf, sem1, sem2 = refs[33:]

    w1_hbm = w_hbm[0:4] + w_hbm[8:12]
    w2_hbm = w_hbm[4:8] + w_hbm[12:16]

    def w1_copy(k):
        return pltpu.make_async_copy(w1_hbm[k], w1buf.at[k % 2], sem1.at[k % 2])

    def w2_copy(k):
        return pltpu.make_async_copy(w2_hbm[k], w2buf.at[k % 2], sem2.at[k % 2])

    w1_copy(0).start()
    w2_copy(0).start()

    x = out0_ref[...]
    coeffs = []
    for k in range(2 * NS):
        if k + 1 < 2 * NS:
            w1_copy(k + 1).start()
            w2_copy(k + 1).start()
        w1_copy(k).wait()
        w2_copy(k).wait()
        w1 = w1buf[k % 2]
        w2 = w2buf[k % 2]

        o = jnp.maximum(x, 0.0)
        if k < NS:
            wsw = wsw_ref[k]
            bsw = bsw_ref[k]
            yl = _dott(o, wsw[0:NB]) + bsw[0:1]
            zm = _dott(o, wsw[NB:2 * NB]) + bsw[1:2]
            zlv = _dott(o, wsw[2 * NB:3 * NB]) + bsw[2:3]
            gl = yl + g_ref[k]
            m = jnp.max(gl, axis=1, keepdims=True)
            iota = lax.broadcasted_iota(jnp.int32, (B, NB), 1)
            idx = jnp.min(jnp.where(gl >= m, iota, NB), axis=1, keepdims=True)
            onehot = (iota == idx).astype(F32)
            z = nz_ref[k] * jnp.exp(zlv * 0.5) + zm
            coeff = onehot * z
            coeffs.append(coeff)
            b1, b2 = b1e_ref[k], b2e_ref[k]
        else:
            coeff = coeffs[k - NS]
            b1, b2 = b1d_ref[k - NS], b2d_ref[k - NS]

        x = x + _mlp_sp(o, coeff, w1, b1, w2, b2)

        if k == NS - 1:
            o = jnp.maximum(x, 0.0)
            z2m = _dott(o, wm_ref[...]) + bm_ref[...]
            z2lv = _dott(o, wv_ref[...]) + bv_ref[...]
            z2 = nz2_ref[...] * jnp.exp(z2lv * 0.5) + z2m
            x = _dott(z2, wl_ref[...]) + bl_ref[...]

    out_ref[...] = jnp.maximum(x, 0.0)


def _conv(x, W, b, stride):
    y = lax.conv_general_dilated(x, W, (stride, stride), ((1, 1), (1, 1)),
                                 dimension_numbers=('NCHW', 'OIHW', 'NCHW'))
    return y + b[None, :, None, None]


def _deconv(x, W, b):
    y = lax.conv_general_dilated(x, jnp.flip(W, (2, 3)), (1, 1), ((2, 2), (2, 2)),
                                 lhs_dilation=(2, 2),
                                 dimension_numbers=('NCHW', 'IOHW', 'NCHW'))
    return y + b[None, :, None, None]


def kernel(x, params):
    key = jax.random.key(42)
    gs, nzs = [], []
    for i in range(NS):
        kg = jax.random.fold_in(key, 2 * i)
        kn = jax.random.fold_in(key, 2 * i + 1)
        gs.append(-jnp.log(jax.random.exponential(kg, (B, NB)) + 1e-20))
        nzs.append(jax.random.normal(kn, (B, NB)))
    g_all = jnp.stack(gs)
    nz_all = jnp.stack(nzs)
    nz2 = jax.random.normal(jax.random.fold_in(key, 999), (B, 10))

    out = jax.nn.relu(_conv(x, params['c1W'], params['c1b'], 2))
    out = jax.nn.relu(_conv(out, params['c2W'], params['c2b'], 2))
    out = jax.nn.relu(_conv(out, params['c3W'], params['c3b'], 2))
    out = _conv(out, params['c4W'], params['c4b'], 2)
    out0 = out.reshape(B, ND)

    enc = params['enc_switches']
    dec = params['dec_switches']
    wsw = jnp.stack([p['Wsw'] for p in enc])
    bsw = jnp.stack([p['bsw'].reshape(3, NB) for p in enc])
    b1e = jnp.stack([p['b1'].reshape(1, ND) for p in enc])
    b2e = jnp.stack([p['b2'] for p in enc])
    b1d = jnp.stack([p['b1'].reshape(1, ND) for p in dec])
    b2d = jnp.stack([p['b2'] for p in dec])

    vmem = pl.BlockSpec(memory_space=pl.ANY)
    d = pl.pallas_call(
        _mega_body,
        in_specs=[pl.BlockSpec()] * 16 + [vmem] * 16,
        out_shape=jax.ShapeDtypeStruct((B, ND), F32),
        scratch_shapes=[
            pltpu.VMEM((2, NB, NDSM, ND), F32),
            pltpu.VMEM((2, NB, ND, NDSM), F32),
            pltpu.SemaphoreType.DMA((2,)),
            pltpu.SemaphoreType.DMA((2,)),
        ],
    )(out0, wsw, bsw, b1e, b2e, b1d, b2d, g_all, nz_all, nz2,
      params['fc_mean_W'], params['fc_mean_b'].reshape(1, 10),
      params['fc_logvar_W'], params['fc_logvar_b'].reshape(1, 10),
      params['fc_latent_W'], params['fc_latent_b'].reshape(1, ND),
      *[p['W1'] for p in enc], *[p['W2'] for p in enc],
      *[p['W1'] for p in dec], *[p['W2'] for p in dec])

    d = d.reshape(B, 64, 4, 4)
    d = jax.nn.relu(_deconv(d, params['d1W'], params['d1b']))
    d = jax.nn.relu(_deconv(d, params['d2W'], params['d2b']))
    d = jax.nn.relu(_deconv(d, params['d3W'], params['d3b']))
    d = _deconv(d, params['d4W'], params['d4b'])
    return d
